# initial kernel scaffold (unmeasured)
import jax
import jax.numpy as jnp
from jax import lax
from jax.experimental import pallas as pl
from jax.experimental.pallas import tpu as pltpu

N_DEV = 4


def kernel(A, B):
    m, k = A.shape
    k2, n = B.shape

    def body(a_ref, b_ref, out_ref, comm_ref, send_sems, recv_sems):
        my_pos = lax.axis_index("i")
        left = (my_pos - 1) % N_DEV
        right = (my_pos + 1) % N_DEV

        barrier_sem = pltpu.get_barrier_semaphore()
        for nbr in [left, right]:
            pl.semaphore_signal(
                barrier_sem, inc=1,
                device_id=(nbr,), device_id_type=pl.DeviceIdType.MESH,
            )
        pl.semaphore_wait(barrier_sem, 2)

        partial = jnp.dot(
            a_ref[...].astype(jnp.bfloat16),
            b_ref[...].astype(jnp.bfloat16),
            preferred_element_type=jnp.float32,
        )
        out_ref[...] = partial
        comm_ref[0, :, :] = partial.astype(jnp.bfloat16)

        for h in range(N_DEV - 1):
            send_slot = h % 2
            recv_slot = (h + 1) % 2
            rdma = pltpu.make_async_remote_copy(
                src_ref=comm_ref.at[send_slot],
                dst_ref=comm_ref.at[recv_slot],
                send_sem=send_sems.at[send_slot],
                recv_sem=recv_sems.at[recv_slot],
                device_id=(right,),
                device_id_type=pl.DeviceIdType.MESH,
            )
            rdma.start()
            rdma.wait()
            out_ref[...] += comm_ref[recv_slot, :, :].astype(jnp.float32)

    return pl.pallas_call(
        body,
        out_shape=jax.ShapeDtypeStruct((m, n), jnp.float32),
        in_specs=[
            pl.BlockSpec(memory_space=pltpu.VMEM),
            pl.BlockSpec(memory_space=pltpu.VMEM),
        ],
        out_specs=pl.BlockSpec(memory_space=pltpu.VMEM),
        scratch_shapes=[
            pltpu.VMEM((2, m, n), jnp.bfloat16),
            pltpu.SemaphoreType.DMA((2,)),
            pltpu.SemaphoreType.DMA((2,)),
        ],
        compiler_params=pltpu.CompilerParams(collective_id=0),
    )(A, B)


# baseline (device time: 16262 ns/iter reference)
import jax
import jax.numpy as jnp
from jax import lax
from jax.experimental import pallas as pl
from jax.experimental.pallas import tpu as pltpu

N_DEV = 4


def kernel(A, B):
    m, k = A.shape
    k2, n = B.shape
    chunk = m // N_DEV

    def body(
        a_ref, b_ref, out_ref,
        stage1_ref, recv1_ref, stage2_ref, recv2_ref,
        s1_sems, r1_sems, s2_sems, r2_sems,
    ):
        my = lax.axis_index("i")

        barrier_sem = pltpu.get_barrier_semaphore()
        for dj in range(1, N_DEV):
            pl.semaphore_signal(
                barrier_sem, inc=1,
                device_id=((my + dj) % N_DEV,),
                device_id_type=pl.DeviceIdType.MESH,
            )
        pl.semaphore_wait(barrier_sem, N_DEV - 1)

        partial = jnp.dot(
            a_ref[...].astype(jnp.bfloat16),
            b_ref[...].astype(jnp.bfloat16),
            preferred_element_type=jnp.float32,
        )
        stage1_ref[...] = partial.reshape(N_DEV, chunk, n).astype(jnp.bfloat16)
        out_ref[...] = partial

        p1 = []
        for dj in range(1, N_DEV):
            peer = (my + dj) % N_DEV
            rdma = pltpu.make_async_remote_copy(
                src_ref=stage1_ref.at[peer],
                dst_ref=recv1_ref.at[dj - 1],
                send_sem=s1_sems.at[dj - 1],
                recv_sem=r1_sems.at[dj - 1],
                device_id=(peer,),
                device_id_type=pl.DeviceIdType.MESH,
            )
            rdma.start()
            p1.append(rdma)
        for rdma in p1:
            rdma.wait_recv()

        red = out_ref[pl.ds(my * chunk, chunk), :]
        for s in range(N_DEV - 1):
            red = red + recv1_ref[s, :, :].astype(jnp.float32)
        out_ref[pl.ds(my * chunk, chunk), :] = red
        stage2_ref[...] = red.astype(jnp.bfloat16)

        p2 = []
        for dj in range(1, N_DEV):
            peer = (my + dj) % N_DEV
            rdma = pltpu.make_async_remote_copy(
                src_ref=stage2_ref,
                dst_ref=recv2_ref.at[dj - 1],
                send_sem=s2_sems.at[dj - 1],
                recv_sem=r2_sems.at[dj - 1],
                device_id=(peer,),
                device_id_type=pl.DeviceIdType.MESH,
            )
            rdma.start()
            p2.append(rdma)
        for s, rdma in enumerate(p2):
            rdma.wait_recv()
            owner = (my - s - 1) % N_DEV
            out_ref[pl.ds(owner * chunk, chunk), :] = (
                recv2_ref[s, :, :].astype(jnp.float32)
            )

        for rdma in p1 + p2:
            rdma.wait_send()

    return pl.pallas_call(
        body,
        out_shape=jax.ShapeDtypeStruct((m, n), jnp.float32),
        in_specs=[
            pl.BlockSpec(memory_space=pltpu.VMEM),
            pl.BlockSpec(memory_space=pltpu.VMEM),
        ],
        out_specs=pl.BlockSpec(memory_space=pltpu.VMEM),
        scratch_shapes=[
            pltpu.VMEM((N_DEV, chunk, n), jnp.bfloat16),
            pltpu.VMEM((N_DEV - 1, chunk, n), jnp.bfloat16),
            pltpu.VMEM((chunk, n), jnp.bfloat16),
            pltpu.VMEM((N_DEV - 1, chunk, n), jnp.bfloat16),
            pltpu.SemaphoreType.DMA((N_DEV - 1,)),
            pltpu.SemaphoreType.DMA((N_DEV - 1,)),
            pltpu.SemaphoreType.DMA((N_DEV - 1,)),
            pltpu.SemaphoreType.DMA((N_DEV - 1,)),
        ],
        compiler_params=pltpu.CompilerParams(collective_id=0),
    )(A, B)


# device time: 16110 ns/iter; 1.0094x vs baseline; 1.0094x over previous
import jax
import jax.numpy as jnp
from jax import lax
from jax.experimental import pallas as pl
from jax.experimental.pallas import tpu as pltpu

N_DEV = 4


def kernel(A, B):
    m, k = A.shape
    k2, n = B.shape
    chunk = m // N_DEV

    def body(
        a_ref, b_ref, out_ref,
        stage1_ref, recv1_ref, stage2_ref, recv2_ref,
        s1_sems, r1_sems, s2_sems, r2_sems,
    ):
        my = lax.axis_index("i")

        barrier_sem = pltpu.get_barrier_semaphore()
        for dj in range(1, N_DEV):
            pl.semaphore_signal(
                barrier_sem, inc=1,
                device_id=((my + dj) % N_DEV,),
                device_id_type=pl.DeviceIdType.MESH,
            )
        pl.semaphore_wait(barrier_sem, N_DEV - 1)

        b16 = b_ref[...].astype(jnp.bfloat16)

        p1 = []
        for dj in range(1, N_DEV):
            peer = (my + dj) % N_DEV
            ck = jnp.dot(
                a_ref[pl.ds(peer * chunk, chunk), :].astype(jnp.bfloat16),
                b16,
                preferred_element_type=jnp.float32,
            )
            stage1_ref[dj - 1, :, :] = ck.astype(jnp.bfloat16)
            rdma = pltpu.make_async_remote_copy(
                src_ref=stage1_ref.at[dj - 1],
                dst_ref=recv1_ref.at[dj - 1],
                send_sem=s1_sems.at[dj - 1],
                recv_sem=r1_sems.at[dj - 1],
                device_id=(peer,),
                device_id_type=pl.DeviceIdType.MESH,
            )
            rdma.start()
            p1.append(rdma)

        red = jnp.dot(
            a_ref[pl.ds(my * chunk, chunk), :].astype(jnp.bfloat16),
            b16,
            preferred_element_type=jnp.float32,
        )

        for s, rdma in enumerate(p1):
            rdma.wait_recv()
            red = red + recv1_ref[s, :, :].astype(jnp.float32)
        out_ref[pl.ds(my * chunk, chunk), :] = red
        stage2_ref[...] = red.astype(jnp.bfloat16)

        p2 = []
        for dj in range(1, N_DEV):
            peer = (my + dj) % N_DEV
            rdma = pltpu.make_async_remote_copy(
                src_ref=stage2_ref,
                dst_ref=recv2_ref.at[dj - 1],
                send_sem=s2_sems.at[dj - 1],
                recv_sem=r2_sems.at[dj - 1],
                device_id=(peer,),
                device_id_type=pl.DeviceIdType.MESH,
            )
            rdma.start()
            p2.append(rdma)
        for s, rdma in enumerate(p2):
            rdma.wait_recv()
            owner = (my - s - 1) % N_DEV
            out_ref[pl.ds(owner * chunk, chunk), :] = (
                recv2_ref[s, :, :].astype(jnp.float32)
            )

        for rdma in p1 + p2:
            rdma.wait_send()

    return pl.pallas_call(
        body,
        out_shape=jax.ShapeDtypeStruct((m, n), jnp.float32),
        in_specs=[
            pl.BlockSpec(memory_space=pltpu.VMEM),
            pl.BlockSpec(memory_space=pltpu.VMEM),
        ],
        out_specs=pl.BlockSpec(memory_space=pltpu.VMEM),
        scratch_shapes=[
            pltpu.VMEM((N_DEV - 1, chunk, n), jnp.bfloat16),
            pltpu.VMEM((N_DEV - 1, chunk, n), jnp.bfloat16),
            pltpu.VMEM((chunk, n), jnp.bfloat16),
            pltpu.VMEM((N_DEV - 1, chunk, n), jnp.bfloat16),
            pltpu.SemaphoreType.DMA((N_DEV - 1,)),
            pltpu.SemaphoreType.DMA((N_DEV - 1,)),
            pltpu.SemaphoreType.DMA((N_DEV - 1,)),
            pltpu.SemaphoreType.DMA((N_DEV - 1,)),
        ],
        compiler_params=pltpu.CompilerParams(collective_id=0),
    )(A, B)


# device time: 15942 ns/iter; 1.0201x vs baseline; 1.0105x over previous
import jax
import jax.numpy as jnp
from jax import lax
from jax.experimental import pallas as pl
from jax.experimental.pallas import tpu as pltpu

N_DEV = 4


def kernel(A, B):
    m, k = A.shape
    k2, n = B.shape
    chunk = m // N_DEV

    def body(
        a_ref, b_ref, out_ref,
        stage1_ref, recv1_ref, stage2_ref,
        s1_sems, r1_sems, s2_sems, r2_sems,
    ):
        my = lax.axis_index("i")

        barrier_sem = pltpu.get_barrier_semaphore()
        for dj in range(1, N_DEV):
            pl.semaphore_signal(
                barrier_sem, inc=1,
                device_id=((my + dj) % N_DEV,),
                device_id_type=pl.DeviceIdType.MESH,
            )
        pl.semaphore_wait(barrier_sem, N_DEV - 1)

        b16 = b_ref[...].astype(jnp.bfloat16)

        p1 = []
        for dj in range(1, N_DEV):
            peer = (my + dj) % N_DEV
            ck = jnp.dot(
                a_ref[pl.ds(peer * chunk, chunk), :].astype(jnp.bfloat16),
                b16,
                preferred_element_type=jnp.float32,
            )
            stage1_ref[dj - 1, :, :] = ck.astype(jnp.bfloat16)
            rdma = pltpu.make_async_remote_copy(
                src_ref=stage1_ref.at[dj - 1],
                dst_ref=recv1_ref.at[dj - 1],
                send_sem=s1_sems.at[dj - 1],
                recv_sem=r1_sems.at[dj - 1],
                device_id=(peer,),
                device_id_type=pl.DeviceIdType.MESH,
            )
            rdma.start()
            p1.append(rdma)

        red = jnp.dot(
            a_ref[pl.ds(my * chunk, chunk), :].astype(jnp.bfloat16),
            b16,
            preferred_element_type=jnp.float32,
        )

        for s, rdma in enumerate(p1):
            rdma.wait_recv()
            red = red + recv1_ref[s, :, :].astype(jnp.float32)
        redb = red.astype(jnp.bfloat16)
        out_ref[pl.ds(my * chunk, chunk), :] = redb
        stage2_ref[...] = redb

        p2 = []
        for dj in range(1, N_DEV):
            peer = (my + dj) % N_DEV
            rdma = pltpu.make_async_remote_copy(
                src_ref=stage2_ref,
                dst_ref=out_ref.at[pl.ds(my * chunk, chunk), :],
                send_sem=s2_sems.at[dj - 1],
                recv_sem=r2_sems.at[dj - 1],
                device_id=(peer,),
                device_id_type=pl.DeviceIdType.MESH,
            )
            rdma.start()
            p2.append(rdma)

        for rdma in p2:
            rdma.wait_recv()
        for rdma in p1 + p2:
            rdma.wait_send()

    return pl.pallas_call(
        body,
        out_shape=jax.ShapeDtypeStruct((m, n), jnp.bfloat16),
        in_specs=[
            pl.BlockSpec(memory_space=pltpu.VMEM),
            pl.BlockSpec(memory_space=pltpu.VMEM),
        ],
        out_specs=pl.BlockSpec(memory_space=pltpu.VMEM),
        scratch_shapes=[
            pltpu.VMEM((N_DEV - 1, chunk, n), jnp.bfloat16),
            pltpu.VMEM((N_DEV - 1, chunk, n), jnp.bfloat16),
            pltpu.VMEM((chunk, n), jnp.bfloat16),
            pltpu.SemaphoreType.DMA((N_DEV - 1,)),
            pltpu.SemaphoreType.DMA((N_DEV - 1,)),
            pltpu.SemaphoreType.DMA((N_DEV - 1,)),
            pltpu.SemaphoreType.DMA((N_DEV - 1,)),
        ],
        compiler_params=pltpu.CompilerParams(collective_id=0),
    )(A, B)


# device time: 14703 ns/iter; 1.1060x vs baseline; 1.0843x over previous
import jax
import jax.numpy as jnp
from jax import lax
from jax.experimental import pallas as pl
from jax.experimental.pallas import tpu as pltpu

N_DEV = 4
N_HALF = 2


def kernel(A, B):
    m, k = A.shape
    k2, n = B.shape
    chunk = m // N_DEV
    n2 = n // N_HALF

    def body(
        a_ref, b_ref, out_ref,
        stage1_ref, recv1_ref, stage2_ref,
        s1_sems, r1_sems, s2_sems, r2_sems,
    ):
        my = lax.axis_index("i")

        barrier_sem = pltpu.get_barrier_semaphore()
        for dj in range(1, N_DEV):
            pl.semaphore_signal(
                barrier_sem, inc=1,
                device_id=((my + dj) % N_DEV,),
                device_id_type=pl.DeviceIdType.MESH,
            )
        pl.semaphore_wait(barrier_sem, N_DEV - 1)

        b16 = b_ref[...].astype(jnp.bfloat16)

        p1 = []
        for h in range(N_HALF):
            for dj in range(1, N_DEV):
                peer = (my + dj) % N_DEV
                ck = jnp.dot(
                    a_ref[pl.ds(peer * chunk, chunk), :].astype(jnp.bfloat16),
                    b16[:, h * n2:(h + 1) * n2],
                    preferred_element_type=jnp.float32,
                )
                stage1_ref[h, dj - 1, :, :] = ck.astype(jnp.bfloat16)
                rdma = pltpu.make_async_remote_copy(
                    src_ref=stage1_ref.at[h, dj - 1],
                    dst_ref=recv1_ref.at[h, dj - 1],
                    send_sem=s1_sems.at[h, dj - 1],
                    recv_sem=r1_sems.at[h, dj - 1],
                    device_id=(peer,),
                    device_id_type=pl.DeviceIdType.MESH,
                )
                rdma.start()
                p1.append(rdma)

        my_ck = jnp.dot(
            a_ref[pl.ds(my * chunk, chunk), :].astype(jnp.bfloat16),
            b16,
            preferred_element_type=jnp.float32,
        )

        p2 = []
        for h in range(N_HALF):
            red = my_ck[:, h * n2:(h + 1) * n2]
            for s in range(N_DEV - 1):
                p1[h * (N_DEV - 1) + s].wait_recv()
                red = red + recv1_ref[h, s, :, :].astype(jnp.float32)
            redb = red.astype(jnp.bfloat16)
            out_ref[pl.ds(my * chunk, chunk), pl.ds(h * n2, n2)] = redb
            stage2_ref[h, :, :] = redb
            for dj in range(1, N_DEV):
                peer = (my + dj) % N_DEV
                rdma = pltpu.make_async_remote_copy(
                    src_ref=stage2_ref.at[h],
                    dst_ref=out_ref.at[pl.ds(my * chunk, chunk),
                                       pl.ds(h * n2, n2)],
                    send_sem=s2_sems.at[h, dj - 1],
                    recv_sem=r2_sems.at[h, dj - 1],
                    device_id=(peer,),
                    device_id_type=pl.DeviceIdType.MESH,
                )
                rdma.start()
                p2.append(rdma)

        for rdma in p2:
            rdma.wait_recv()
        for rdma in p1 + p2:
            rdma.wait_send()

    return pl.pallas_call(
        body,
        out_shape=jax.ShapeDtypeStruct((m, n), jnp.bfloat16),
        in_specs=[
            pl.BlockSpec(memory_space=pltpu.VMEM),
            pl.BlockSpec(memory_space=pltpu.VMEM),
        ],
        out_specs=pl.BlockSpec(memory_space=pltpu.VMEM),
        scratch_shapes=[
            pltpu.VMEM((N_HALF, N_DEV - 1, chunk, n2), jnp.bfloat16),
            pltpu.VMEM((N_HALF, N_DEV - 1, chunk, n2), jnp.bfloat16),
            pltpu.VMEM((N_HALF, chunk, n2), jnp.bfloat16),
            pltpu.SemaphoreType.DMA((N_HALF, N_DEV - 1)),
            pltpu.SemaphoreType.DMA((N_HALF, N_DEV - 1)),
            pltpu.SemaphoreType.DMA((N_HALF, N_DEV - 1)),
            pltpu.SemaphoreType.DMA((N_HALF, N_DEV - 1)),
        ],
        compiler_params=pltpu.CompilerParams(collective_id=0),
    )(A, B)


# device time: 6812 ns/iter; 2.3873x vs baseline; 2.1584x over previous
import jax
import jax.numpy as jnp
from jax import lax
from jax.experimental import pallas as pl
from jax.experimental.pallas import tpu as pltpu

N_DEV = 4
N_HALF = 2
_DJ_ORDER = (2, 1, 3)


def kernel(A, B):
    m, k = A.shape
    k2, n = B.shape
    chunk = m // N_DEV
    n2 = n // N_HALF

    def body(
        a_ref, b_ref, out_ref,
        stage1_ref, recv1_ref,
        s1_sems, r1_sems, s2_sems, r2_sems,
    ):
        my = lax.axis_index("i")

        barrier_sem = pltpu.get_barrier_semaphore()
        for dj in range(1, N_DEV):
            pl.semaphore_signal(
                barrier_sem, inc=1,
                device_id=((my + dj) % N_DEV,),
                device_id_type=pl.DeviceIdType.MESH,
            )

        b16 = b_ref[...].astype(jnp.bfloat16)

        def compute_half(h):
            for dj in _DJ_ORDER:
                peer = (my + dj) % N_DEV
                ck = jnp.dot(
                    a_ref[pl.ds(peer * chunk, chunk), :].astype(jnp.bfloat16),
                    b16[:, h * n2:(h + 1) * n2],
                    preferred_element_type=jnp.float32,
                )
                stage1_ref[h, dj - 1, :, :] = ck.astype(jnp.bfloat16)

        def send_half(h):
            rdmas = {}
            for dj in _DJ_ORDER:
                peer = (my + dj) % N_DEV
                rdma = pltpu.make_async_remote_copy(
                    src_ref=stage1_ref.at[h, dj - 1],
                    dst_ref=recv1_ref.at[h, dj - 1],
                    send_sem=s1_sems.at[h, dj - 1],
                    recv_sem=r1_sems.at[h, dj - 1],
                    device_id=(peer,),
                    device_id_type=pl.DeviceIdType.MESH,
                )
                rdma.start()
                rdmas[dj] = rdma
            return rdmas

        compute_half(0)
        pl.semaphore_wait(barrier_sem, N_DEV - 1)
        p1 = {0: send_half(0)}
        compute_half(1)
        p1[1] = send_half(1)

        my_ck = jnp.dot(
            a_ref[pl.ds(my * chunk, chunk), :].astype(jnp.bfloat16),
            b16,
            preferred_element_type=jnp.float32,
        )

        p2 = []
        for h in range(N_HALF):
            red = my_ck[:, h * n2:(h + 1) * n2]
            for dj in range(1, N_DEV):
                p1[h][dj].wait_recv()
                red = red + recv1_ref[h, dj - 1, :, :].astype(jnp.float32)
            out_ref[pl.ds(my * chunk, chunk), pl.ds(h * n2, n2)] = (
                red.astype(jnp.bfloat16)
            )
            for dj in _DJ_ORDER:
                peer = (my + dj) % N_DEV
                rdma = pltpu.make_async_remote_copy(
                    src_ref=out_ref.at[pl.ds(my * chunk, chunk),
                                       pl.ds(h * n2, n2)],
                    dst_ref=out_ref.at[pl.ds(my * chunk, chunk),
                                       pl.ds(h * n2, n2)],
                    send_sem=s2_sems.at[h, dj - 1],
                    recv_sem=r2_sems.at[h, dj - 1],
                    device_id=(peer,),
                    device_id_type=pl.DeviceIdType.MESH,
                )
                rdma.start()
                p2.append(rdma)

        for rdma in p2:
            rdma.wait_recv()
        for rdmas in p1.values():
            for rdma in rdmas.values():
                rdma.wait_send()
        for rdma in p2:
            rdma.wait_send()

    return pl.pallas_call(
        body,
        out_shape=jax.ShapeDtypeStruct((m, n), jnp.bfloat16),
        in_specs=[
            pl.BlockSpec(memory_space=pltpu.VMEM),
            pl.BlockSpec(memory_space=pltpu.VMEM),
        ],
        out_specs=pl.BlockSpec(memory_space=pltpu.VMEM),
        scratch_shapes=[
            pltpu.VMEM((N_HALF, N_DEV - 1, chunk, n2), jnp.bfloat16),
            pltpu.VMEM((N_HALF, N_DEV - 1, chunk, n2), jnp.bfloat16),
            pltpu.SemaphoreType.DMA((N_HALF, N_DEV - 1)),
            pltpu.SemaphoreType.DMA((N_HALF, N_DEV - 1)),
            pltpu.SemaphoreType.DMA((N_HALF, N_DEV - 1)),
            pltpu.SemaphoreType.DMA((N_HALF, N_DEV - 1)),
        ],
        compiler_params=pltpu.CompilerParams(collective_id=0),
    )(A, B)
